# baseline (speedup 1.0000x reference)
import functools

import jax
import jax.numpy as jnp
from jax import lax
from jax.experimental import pallas as pl
from jax.experimental.pallas import tpu as pltpu
from jax.experimental.pallas import tpu_sc as plsc

NUM_EXPERTS = 64
TOP_K = 2
HIDDEN = 2048
TOKENS = 16384

BT = 2048

NC = 2
NS = 16
LANES = 16
NW = NC * NS
TPW = TOKENS // NW


def _gate_block(x_ref, w_ref, p_ref):
    logits = lax.dot_general(
        w_ref[...], x_ref[...],
        dimension_numbers=(((1,), (1,)), ((), ())),
        preferred_element_type=jnp.float32,
        precision=lax.Precision.DEFAULT,
    )
    m1 = jnp.max(logits, axis=0, keepdims=True)
    u = jnp.exp(logits - m1)
    p_ref[...] = u / jnp.sum(u, axis=0, keepdims=True)


NCHUNK = 4
CHUNK = TOKENS // NCHUNK
CPW = CHUNK // NW


def _top2_sc(p_hbm, w_out, i_out, slab, wbuf, ibuf):
    c = lax.axis_index("c")
    s = lax.axis_index("s")
    wid = s * NC + c
    base = wid * CPW
    pltpu.sync_copy(p_hbm.at[:, pl.ds(base, CPW)], slab)

    def group(g, carry):
        off = pl.multiple_of(g * LANES, LANES)
        p1 = slab[0, pl.ds(off, LANES)]
        i1 = jnp.zeros((LANES,), jnp.int32)
        p2 = jnp.full((LANES,), -1.0, jnp.float32)
        i2 = jnp.zeros((LANES,), jnp.int32)
        for e in range(1, NUM_EXPERTS):
            pe = slab[e, pl.ds(off, LANES)]
            ei = jnp.full((LANES,), e, jnp.int32)
            gt1 = pe > p1
            gt2 = pe > p2
            p2 = jnp.maximum(p2, jnp.minimum(pe, p1))
            i2 = jnp.where(gt1, i1, jnp.where(gt2, ei, i2))
            p1 = jnp.maximum(p1, pe)
            i1 = jnp.where(gt1, ei, i1)
        wbuf[0, pl.ds(off, LANES)] = p1
        wbuf[1, pl.ds(off, LANES)] = p2
        ibuf[0, pl.ds(off, LANES)] = i1
        ibuf[1, pl.ds(off, LANES)] = i2
        return carry

    lax.fori_loop(0, CPW // LANES, group, 0)
    pltpu.sync_copy(wbuf, w_out.at[:, pl.ds(base, CPW)])
    pltpu.sync_copy(ibuf, i_out.at[:, pl.ds(base, CPW)])


@jax.jit
def kernel(x, weight):
    mesh = plsc.VectorSubcoreMesh(
        core_axis_name="c", subcore_axis_name="s",
        num_cores=NC, num_subcores=NS,
    )
    sc_top2 = pl.kernel(
        _top2_sc,
        out_type=[
            jax.ShapeDtypeStruct((TOP_K, CHUNK), jnp.float32),
            jax.ShapeDtypeStruct((TOP_K, CHUNK), jnp.int32),
        ],
        mesh=mesh,
        scratch_types=[
            pltpu.VMEM((NUM_EXPERTS, CPW), jnp.float32),
            pltpu.VMEM((TOP_K, CPW), jnp.float32),
            pltpu.VMEM((TOP_K, CPW), jnp.int32),
        ],
    )

    wts, its = [], []
    for ci in range(NCHUNK):
        probs_t = pl.pallas_call(
            _gate_block,
            grid=(CHUNK // BT,),
            in_specs=[
                pl.BlockSpec((BT, HIDDEN),
                             lambda i, _c=ci: (_c * (CHUNK // BT) + i, 0)),
                pl.BlockSpec((NUM_EXPERTS, HIDDEN), lambda i: (0, 0)),
            ],
            out_specs=pl.BlockSpec((NUM_EXPERTS, BT), lambda i: (0, i)),
            out_shape=jax.ShapeDtypeStruct((NUM_EXPERTS, CHUNK), jnp.float32),
        )(x, weight)
        wt, it = sc_top2(probs_t)
        wts.append(wt)
        its.append(it)

    wt = jnp.concatenate(wts, axis=1)
    it = jnp.concatenate(its, axis=1)
    return wt.T, it.T


# speedup vs baseline: 1.2078x; 1.2078x over previous
import functools

import jax
import jax.numpy as jnp
from jax import lax
from jax.experimental import pallas as pl
from jax.experimental.pallas import tpu as pltpu
from jax.experimental.pallas import tpu_sc as plsc

NUM_EXPERTS = 64
TOP_K = 2
HIDDEN = 2048
TOKENS = 16384

BT = 2048

NC = 2
NS = 16
LANES = 16
NW = NC * NS
TPW = TOKENS // NW


def _gate_block(x_ref, w_ref, p_ref):
    logits = lax.dot_general(
        w_ref[...], x_ref[...],
        dimension_numbers=(((1,), (1,)), ((), ())),
        preferred_element_type=jnp.float32,
        precision=lax.Precision.DEFAULT,
    )
    m1 = jnp.max(logits, axis=0, keepdims=True)
    u = jnp.exp(logits - m1)
    p_ref[...] = u / jnp.sum(u, axis=0, keepdims=True)


NBUF = 2
HPW = TPW // NBUF


def _top2_sc(p_hbm, w_out, i_out, slab0, slab1, wbuf, ibuf, sem0, sem1):
    c = lax.axis_index("c")
    s = lax.axis_index("s")
    wid = s * NC + c
    base = wid * TPW
    cp0 = pltpu.make_async_copy(p_hbm.at[:, pl.ds(base, HPW)], slab0, sem0)
    cp1 = pltpu.make_async_copy(p_hbm.at[:, pl.ds(base + HPW, HPW)], slab1, sem1)
    cp0.start()
    cp1.start()

    def make_group(slab, obase):
        def group(g, carry):
            off = pl.multiple_of(g * LANES, LANES)
            p1 = slab[0, pl.ds(off, LANES)]
            i1 = jnp.zeros((LANES,), jnp.int32)
            p2 = jnp.full((LANES,), -1.0, jnp.float32)
            i2 = jnp.zeros((LANES,), jnp.int32)
            for e in range(1, NUM_EXPERTS):
                pe = slab[e, pl.ds(off, LANES)]
                ei = jnp.full((LANES,), e, jnp.int32)
                gt1 = pe > p1
                gt2 = pe > p2
                p2 = jnp.maximum(p2, jnp.minimum(pe, p1))
                i2 = jnp.where(gt1, i1, jnp.where(gt2, ei, i2))
                p1 = jnp.maximum(p1, pe)
                i1 = jnp.where(gt1, ei, i1)
            oof = pl.multiple_of(obase + g * LANES, LANES)
            wbuf[0, pl.ds(oof, LANES)] = p1
            wbuf[1, pl.ds(oof, LANES)] = p2
            ibuf[0, pl.ds(oof, LANES)] = i1
            ibuf[1, pl.ds(oof, LANES)] = i2
            return carry
        return group

    cp0.wait()
    lax.fori_loop(0, HPW // LANES, make_group(slab0, 0), 0)
    cp1.wait()
    lax.fori_loop(0, HPW // LANES, make_group(slab1, HPW), 0)
    pltpu.sync_copy(wbuf, w_out.at[:, pl.ds(base, TPW)])
    pltpu.sync_copy(ibuf, i_out.at[:, pl.ds(base, TPW)])


@jax.jit
def kernel(x, weight):
    probs_t = pl.pallas_call(
        _gate_block,
        grid=(TOKENS // BT,),
        in_specs=[
            pl.BlockSpec((BT, HIDDEN), lambda i: (i, 0)),
            pl.BlockSpec((NUM_EXPERTS, HIDDEN), lambda i: (0, 0)),
        ],
        out_specs=pl.BlockSpec((NUM_EXPERTS, BT), lambda i: (0, i)),
        out_shape=jax.ShapeDtypeStruct((NUM_EXPERTS, TOKENS), jnp.float32),
    )(x, weight)

    mesh = plsc.VectorSubcoreMesh(
        core_axis_name="c", subcore_axis_name="s",
        num_cores=NC, num_subcores=NS,
    )
    wt, it = pl.kernel(
        _top2_sc,
        out_type=[
            jax.ShapeDtypeStruct((TOP_K, TOKENS), jnp.float32),
            jax.ShapeDtypeStruct((TOP_K, TOKENS), jnp.int32),
        ],
        mesh=mesh,
        scratch_types=[
            pltpu.VMEM((NUM_EXPERTS, HPW), jnp.float32),
            pltpu.VMEM((NUM_EXPERTS, HPW), jnp.float32),
            pltpu.VMEM((TOP_K, TPW), jnp.float32),
            pltpu.VMEM((TOP_K, TPW), jnp.int32),
            pltpu.SemaphoreType.DMA,
            pltpu.SemaphoreType.DMA,
        ],
    )(probs_t)
    return wt.T, it.T
